# baseline (speedup 1.0000x reference)
import jax
import jax.numpy as jnp
from jax.experimental import pallas as pl
from jax.experimental.pallas import tpu as pltpu

TOPK = 8
N_GROUPS = 8
TOPK_GROUPS = 4
ROUTE_SCALE = 2.5
N_EXPERTS = 256
GROUP_SIZE = N_EXPERTS // N_GROUPS

BM = 512


def _routing_epilogue(acc_ref, b_ref, wout_ref, iout_ref):
    neg_inf = jnp.float32(-jnp.inf)
    s = jax.nn.sigmoid(acc_ref[...].T)
    sb = s + b_ref[...]
    it = jax.lax.broadcasted_iota(jnp.int32, sb.shape, 0)

    gsc = []
    for g in range(N_GROUPS):
        sg = sb[g * GROUP_SIZE:(g + 1) * GROUP_SIZE, :]
        m1 = jnp.max(sg, axis=0, keepdims=True)
        cnt = jnp.sum((sg == m1).astype(jnp.float32), axis=0, keepdims=True)
        m2 = jnp.max(jnp.where(sg < m1, sg, neg_inf), axis=0, keepdims=True)
        gsc.append(m1 + jnp.where(cnt > 1.5, m1, m2))

    rank = [jnp.zeros_like(gsc[0]) for _ in range(N_GROUPS)]
    for g in range(N_GROUPS):
        for h in range(g + 1, N_GROUPS):
            c = (gsc[h] > gsc[g]).astype(jnp.float32)
            rank[g] = rank[g] + c
            rank[h] = rank[h] + (1.0 - c)
    masked = jnp.concatenate(
        [jnp.where(rank[g] < float(TOPK_GROUPS),
                   sb[g * GROUP_SIZE:(g + 1) * GROUP_SIZE, :], neg_inf)
         for g in range(N_GROUPS)], axis=0)

    widx = []
    wval = []
    for j in range(TOPK):
        m = jnp.max(masked, axis=0, keepdims=True)
        idx = jnp.min(jnp.where(masked == m, it, N_EXPERTS),
                      axis=0, keepdims=True)
        sel = it == idx
        v = jnp.max(jnp.where(sel, s, neg_inf), axis=0, keepdims=True)
        widx.append(idx)
        wval.append(v)
        masked = jnp.where(sel, neg_inf, masked)

    wvalt = jnp.concatenate(wval, axis=0)
    widxt = jnp.concatenate(widx, axis=0)
    wsum = wval[0]
    for j in range(1, TOPK):
        wsum = wsum + wval[j]
    wout_ref[...] = (wvalt / wsum * ROUTE_SCALE).T
    iout_ref[...] = widxt.T


def _gate_kernel(x_ref, w_ref, b_ref, wout_ref, iout_ref, acc_ref):
    _routing_epilogue(acc_ref, b_ref, wout_ref, iout_ref)

    mm = jax.lax.dot_general(
        x_ref[...], w_ref[...],
        dimension_numbers=(((1,), (1,)), ((), ())),
        preferred_element_type=jnp.float32)
    acc_ref[...] = mm


@jax.jit
def kernel(x, weight, bias):
    B, K = x.shape
    n_m = B // BM
    b2 = bias.astype(jnp.float32).reshape(N_EXPERTS, 1)
    woutt, ioutt = pl.pallas_call(
        _gate_kernel,
        grid=(n_m + 1,),
        in_specs=[
            pl.BlockSpec((BM, K), lambda i: (jnp.minimum(i, n_m - 1), 0)),
            pl.BlockSpec((N_EXPERTS, K), lambda i: (0, 0)),
            pl.BlockSpec((N_EXPERTS, 1), lambda i: (0, 0)),
        ],
        out_specs=[
            pl.BlockSpec((BM, TOPK), lambda i: (jnp.maximum(i - 1, 0), 0)),
            pl.BlockSpec((BM, TOPK), lambda i: (jnp.maximum(i - 1, 0), 0)),
        ],
        out_shape=[
            jax.ShapeDtypeStruct((B, TOPK), jnp.float32),
            jax.ShapeDtypeStruct((B, TOPK), jnp.int32),
        ],
        scratch_shapes=[
            pltpu.VMEM((BM, N_EXPERTS), jnp.float32),
        ],
        compiler_params=pltpu.CompilerParams(
            dimension_semantics=("arbitrary",),
        ),
    )(x.astype(jnp.float32), weight.astype(jnp.float32), b2)
    return woutt, ioutt


# speedup vs baseline: 1.0852x; 1.0852x over previous
import jax
import jax.numpy as jnp
from jax.experimental import pallas as pl
from jax.experimental.pallas import tpu as pltpu

TOPK = 8
N_GROUPS = 8
TOPK_GROUPS = 4
ROUTE_SCALE = 2.5
N_EXPERTS = 256
GROUP_SIZE = N_EXPERTS // N_GROUPS

BM = 512


def _routing_epilogue(acc_ref, b_ref, wout_ref, iout_ref):
    neg_inf = jnp.float32(-jnp.inf)
    s = jax.nn.sigmoid(acc_ref[...].T)
    sb = s + b_ref[...]
    it = jax.lax.broadcasted_iota(jnp.int32, sb.shape, 0)

    gsc = []
    for g in range(N_GROUPS):
        sg = sb[g * GROUP_SIZE:(g + 1) * GROUP_SIZE, :]
        m1 = jnp.max(sg, axis=0, keepdims=True)
        cnt = jnp.sum((sg == m1).astype(jnp.float32), axis=0, keepdims=True)
        m2 = jnp.max(jnp.where(sg < m1, sg, neg_inf), axis=0, keepdims=True)
        gsc.append(m1 + jnp.where(cnt > 1.5, m1, m2))

    rank = [jnp.zeros_like(gsc[0]) for _ in range(N_GROUPS)]
    for g in range(N_GROUPS):
        for h in range(g + 1, N_GROUPS):
            c = (gsc[h] > gsc[g]).astype(jnp.float32)
            rank[g] = rank[g] + c
            rank[h] = rank[h] + (1.0 - c)
    masked = jnp.concatenate(
        [jnp.where(rank[g] < float(TOPK_GROUPS),
                   sb[g * GROUP_SIZE:(g + 1) * GROUP_SIZE, :], neg_inf)
         for g in range(N_GROUPS)], axis=0)

    widx = []
    wval = []
    for j in range(TOPK):
        m = jnp.max(masked, axis=0, keepdims=True)
        idx = jnp.min(jnp.where(masked == m, it, N_EXPERTS),
                      axis=0, keepdims=True)
        sel = it == idx
        v = jnp.max(jnp.where(sel, s, neg_inf), axis=0, keepdims=True)
        widx.append(idx)
        wval.append(v)
        masked = jnp.where(sel, neg_inf, masked)

    wvalt = jnp.concatenate(wval, axis=0)
    widxt = jnp.concatenate(widx, axis=0)
    wsum = wval[0]
    for j in range(1, TOPK):
        wsum = wsum + wval[j]
    wout_ref[...] = wvalt / wsum * ROUTE_SCALE
    iout_ref[...] = widxt


def _gate_kernel(x_ref, w_ref, b_ref, wout_ref, iout_ref, acc_ref):
    _routing_epilogue(acc_ref, b_ref, wout_ref, iout_ref)

    mm = jax.lax.dot_general(
        x_ref[...], w_ref[...],
        dimension_numbers=(((1,), (1,)), ((), ())),
        preferred_element_type=jnp.float32)
    acc_ref[...] = mm


@jax.jit
def kernel(x, weight, bias):
    B, K = x.shape
    n_m = B // BM
    b2 = bias.astype(jnp.float32).reshape(N_EXPERTS, 1)
    woutt, ioutt = pl.pallas_call(
        _gate_kernel,
        grid=(n_m + 1,),
        in_specs=[
            pl.BlockSpec((BM, K), lambda i: (jnp.minimum(i, n_m - 1), 0)),
            pl.BlockSpec((N_EXPERTS, K), lambda i: (0, 0)),
            pl.BlockSpec((N_EXPERTS, 1), lambda i: (0, 0)),
        ],
        out_specs=[
            pl.BlockSpec((TOPK, BM), lambda i: (0, jnp.maximum(i - 1, 0))),
            pl.BlockSpec((TOPK, BM), lambda i: (0, jnp.maximum(i - 1, 0))),
        ],
        out_shape=[
            jax.ShapeDtypeStruct((TOPK, B), jnp.float32),
            jax.ShapeDtypeStruct((TOPK, B), jnp.int32),
        ],
        scratch_shapes=[
            pltpu.VMEM((BM, N_EXPERTS), jnp.float32),
        ],
        compiler_params=pltpu.CompilerParams(
            dimension_semantics=("arbitrary",),
        ),
    )(x.astype(jnp.float32), weight.astype(jnp.float32), b2)
    return woutt.T, ioutt.T
